# Initial kernel scaffold; baseline (speedup 1.0000x reference)
import jax
import jax.numpy as jnp
from jax.experimental import pallas as pl


def kernel(features, text, atten, linear_W, linear_b, mlp_W1, mlp_b1, bn_gamma, bn_beta, mlp_W2, mlp_b2):
    raise NotImplementedError("write your pallas kernel here")


# speedup vs baseline: 1.2321x; 1.2321x over previous
import functools
import jax
import jax.numpy as jnp
from jax.experimental import pallas as pl
from jax.experimental.pallas import tpu as pltpu

_RATIO = 0.3


def _argmax_kernel(text_ref, am_ref):
    t = text_ref[...]
    b, l = t.shape
    m = jnp.max(t, axis=1, keepdims=True)
    iota = jax.lax.broadcasted_iota(jnp.int32, (b, l), 1)
    am_ref[...] = jnp.min(jnp.where(t == m, iota, l), axis=1, keepdims=True)


def _argmax_call(text):
    B, L = text.shape
    return pl.pallas_call(
        _argmax_kernel,
        out_shape=jax.ShapeDtypeStruct((B, 1), jnp.int32),
    )(text)


def _extract_kernel(am_ref, atten_ref, text_ref, rows_ref):
    b = pl.program_id(0)
    a = am_ref[b]
    row = atten_ref[0, pl.ds(a % 8, 1), :]
    t = text_ref[...].reshape(1, text_ref.shape[-1])
    iota = jax.lax.broadcasted_iota(jnp.int32, row.shape, 1)
    row = jnp.where(iota == a, -1.0, row)
    row = jnp.where(t != 0, row, 0.0)
    rows_ref[...] = row.reshape(rows_ref.shape)


def _extract_call(am, atten, text3):
    B, L, _ = atten.shape
    grid_spec = pltpu.PrefetchScalarGridSpec(
        num_scalar_prefetch=1,
        grid=(B,),
        in_specs=[
            pl.BlockSpec((1, 8, L), lambda b, am_s: (b, am_s[b] // 8, 0)),
            pl.BlockSpec((1, 1, L), lambda b, am_s: (b, 0, 0)),
        ],
        out_specs=pl.BlockSpec((1, 1, L), lambda b, am_s: (b, 0, 0)),
    )
    return pl.pallas_call(
        _extract_kernel,
        grid_spec=grid_spec,
        out_shape=jax.ShapeDtypeStruct((B, 1, L), jnp.float32),
    )(am, atten, text3)


def _topk_kernel(rows_ref, idx_ref, *, k, k_pad):
    v = rows_ref[...]
    B, L = v.shape
    ix = jax.lax.broadcasted_iota(jnp.int32, (B, L), 1)
    ci = jax.lax.broadcasted_iota(jnp.int32, (1, L), 1)

    kk = 2
    while kk <= L:
        j = kk // 2
        while j >= 1:
            bit = (ci & j) != 0
            desc = (ci & kk) == 0
            if kk == L:
                desc = jnp.full_like(bit, True)
            pv = jnp.where(bit, jnp.roll(v, j, axis=1), jnp.roll(v, -j, axis=1))
            px = jnp.where(bit, jnp.roll(ix, j, axis=1), jnp.roll(ix, -j, axis=1))
            lo_v = jnp.where(bit, pv, v)
            hi_v = jnp.where(bit, v, pv)
            lo_i = jnp.where(bit, px, ix)
            hi_i = jnp.where(bit, ix, px)
            good = (lo_v > hi_v) | ((lo_v == hi_v) & (lo_i < hi_i))
            swap = jnp.logical_xor(good, desc)
            v = jnp.where(swap, pv, v)
            ix = jnp.where(swap, px, ix)
            j //= 2
        kk *= 2

    out = ix[:, :k_pad]
    cpad = jax.lax.broadcasted_iota(jnp.int32, (1, k_pad), 1)
    idx_ref[...] = jnp.where(cpad < k, out, -1)


def _topk_call(rows, k, k_pad):
    B, L = rows.shape
    return pl.pallas_call(
        functools.partial(_topk_kernel, k=k, k_pad=k_pad),
        out_shape=jax.ShapeDtypeStruct((B, k_pad), jnp.int32),
    )(rows)


def _gather_stats_kernel(feat_ref, tk_ref, xn_ref, g_ref, s_ref,
                         g_acc, s_acc):
    b = pl.program_id(0)
    nb = pl.num_programs(0)
    f = feat_ref[0]
    tk = tk_ref[0]
    L, D = f.shape
    kp = tk.shape[-1]
    cc = jax.lax.broadcasted_iota(jnp.int32, (L, kp), 0)
    pt = (cc == tk).astype(jnp.float32)
    f_hi = f.astype(jnp.bfloat16).astype(jnp.float32)
    f_lo = f - f_hi
    x = (jax.lax.dot_general(pt, f_hi, (((0,), (0,)), ((), ())),
                             preferred_element_type=jnp.float32)
         + jax.lax.dot_general(pt, f_lo, (((0,), (0,)), ((), ())),
                               preferred_element_type=jnp.float32))
    norm = jnp.sqrt(jnp.sum(x * x, axis=1, keepdims=True)) + 1e-8
    xn = x / norm
    xn_ref[0] = xn

    g_step = jax.lax.dot_general(xn, xn, (((0,), (0,)), ((), ())),
                                 preferred_element_type=jnp.float32)
    s_step = jnp.sum(xn, axis=0, keepdims=True)

    @pl.when(b == 0)
    def _():
        g_acc[...] = jnp.zeros_like(g_acc)
        s_acc[...] = jnp.zeros_like(s_acc)

    g_acc[...] += g_step
    s_acc[...] += s_step

    @pl.when(b == nb - 1)
    def _():
        g_ref[...] = g_acc[...]
        s_ref[...] = s_acc[...]


def _gather_stats_call(features, tk3, k_pad):
    B, L, D = features.shape
    grid = (B,)
    return pl.pallas_call(
        _gather_stats_kernel,
        grid=grid,
        in_specs=[
            pl.BlockSpec((1, L, D), lambda b: (b, 0, 0)),
            pl.BlockSpec((1, 1, k_pad), lambda b: (b, 0, 0)),
        ],
        out_specs=[
            pl.BlockSpec((1, k_pad, D), lambda b: (b, 0, 0)),
            pl.BlockSpec((D, D), lambda b: (0, 0)),
            pl.BlockSpec((1, D), lambda b: (0, 0)),
        ],
        out_shape=[
            jax.ShapeDtypeStruct((B, k_pad, D), jnp.float32),
            jax.ShapeDtypeStruct((D, D), jnp.float32),
            jax.ShapeDtypeStruct((1, D), jnp.float32),
        ],
        scratch_shapes=[
            pltpu.VMEM((D, D), jnp.float32),
            pltpu.VMEM((1, D), jnp.float32),
        ],
    )(features, tk3)


def _final_kernel(xn_ref, g_ref, s_ref, lw_ref, lb_ref, w1_ref, b1_ref,
                  gam_ref, bet_ref, w2_ref, b2_ref, out_ref,
                  scale_scr, shift_scr, *, n_rows):
    b = pl.program_id(0)

    @pl.when(b == 0)
    def _():
        w1 = w1_ref[...]
        s = s_ref[...] / n_rows
        g = g_ref[...] / n_rows
        mean_wx = jax.lax.dot_general(s, w1, (((1,), (1,)), ((), ())),
                                      preferred_element_type=jnp.float32)
        m = jax.lax.dot_general(w1, g, (((1,), (0,)), ((), ())),
                                preferred_element_type=jnp.float32)
        ones = jnp.ones((1, m.shape[1]), jnp.float32)
        e = jax.lax.dot_general(ones, m * w1, (((1,), (1,)), ((), ())),
                                preferred_element_type=jnp.float32)
        var = e - mean_wx * mean_wx
        mean_h = mean_wx + b1_ref[...]
        scale = gam_ref[...] * jax.lax.rsqrt(var + 1e-5)
        scale_scr[...] = scale
        shift_scr[...] = bet_ref[...] - mean_h * scale

    x = xn_ref[0]
    cap = jax.lax.dot_general(x, lw_ref[...], (((1,), (1,)), ((), ())),
                              preferred_element_type=jnp.float32) + lb_ref[...]
    h = jax.lax.dot_general(x, w1_ref[...], (((1,), (1,)), ((), ())),
                            preferred_element_type=jnp.float32) + b1_ref[...]
    h = h * scale_scr[...] + shift_scr[...]
    h = jnp.maximum(h, 0.0)
    out = jax.lax.dot_general(h, w2_ref[...], (((1,), (1,)), ((), ())),
                              preferred_element_type=jnp.float32) + b2_ref[...]
    out_ref[0] = out + cap


def _final_call(xn, g, s, lw, lb, w1, b1, gam, bet, w2, b2, n_rows):
    B, k_pad, D = xn.shape
    DE = lw.shape[0]
    H = w1.shape[0]
    grid = (B,)
    return pl.pallas_call(
        functools.partial(_final_kernel, n_rows=float(n_rows)),
        grid=grid,
        in_specs=[
            pl.BlockSpec((1, k_pad, D), lambda b: (b, 0, 0)),
            pl.BlockSpec((D, D), lambda b: (0, 0)),
            pl.BlockSpec((1, D), lambda b: (0, 0)),
            pl.BlockSpec((DE, D), lambda b: (0, 0)),
            pl.BlockSpec((1, DE), lambda b: (0, 0)),
            pl.BlockSpec((H, D), lambda b: (0, 0)),
            pl.BlockSpec((1, H), lambda b: (0, 0)),
            pl.BlockSpec((1, H), lambda b: (0, 0)),
            pl.BlockSpec((1, H), lambda b: (0, 0)),
            pl.BlockSpec((DE, H), lambda b: (0, 0)),
            pl.BlockSpec((1, DE), lambda b: (0, 0)),
        ],
        out_specs=pl.BlockSpec((1, k_pad, DE), lambda b: (b, 0, 0)),
        out_shape=jax.ShapeDtypeStruct((B, k_pad, DE), jnp.float32),
        scratch_shapes=[
            pltpu.VMEM((1, H), jnp.float32),
            pltpu.VMEM((1, H), jnp.float32),
        ],
    )(xn, g, s, lw, lb, w1, b1, gam, bet, w2, b2)


def kernel(features, text, atten, linear_W, linear_b, mlp_W1, mlp_b1,
           bn_gamma, bn_beta, mlp_W2, mlp_b2):
    B, L, D = features.shape
    k = int((atten.shape[1] - 2) * _RATIO)
    k_pad = ((k + 7) // 8) * 8

    am = _argmax_call(text).reshape(B)
    rows = _extract_call(am, atten, text.reshape(B, 1, L)).reshape(B, L)
    tk = _topk_call(rows, k, k_pad)
    tk3 = tk.reshape(B, 1, k_pad)
    xn, g, s = _gather_stats_call(features, tk3, k_pad)
    out = _final_call(xn, g, s, linear_W, linear_b.reshape(1, -1),
                      mlp_W1, mlp_b1.reshape(1, -1),
                      bn_gamma.reshape(1, -1), bn_beta.reshape(1, -1),
                      mlp_W2, mlp_b2.reshape(1, -1), n_rows=B * k)
    return out[:, :k, :]


# speedup vs baseline: 1.2487x; 1.0135x over previous
import functools
import jax
import jax.numpy as jnp
from jax.experimental import pallas as pl
from jax.experimental.pallas import tpu as pltpu

_RATIO = 0.3


def _argmax_kernel(text_ref, am_ref):
    t = text_ref[...]
    b, l = t.shape
    m = jnp.max(t, axis=1, keepdims=True)
    iota = jax.lax.broadcasted_iota(jnp.int32, (b, l), 1)
    am_ref[...] = jnp.min(jnp.where(t == m, iota, l), axis=1, keepdims=True)


def _argmax_call(text):
    B, L = text.shape
    return pl.pallas_call(
        _argmax_kernel,
        out_shape=jax.ShapeDtypeStruct((B, 1), jnp.int32),
    )(text)


def _extract_kernel(am_ref, atten_ref, text_ref, rows_ref):
    b = pl.program_id(0)
    a = am_ref[b]
    row = atten_ref[0, pl.ds(a % 8, 1), :]
    t = text_ref[...].reshape(1, text_ref.shape[-1])
    iota = jax.lax.broadcasted_iota(jnp.int32, row.shape, 1)
    row = jnp.where(iota == a, -1.0, row)
    row = jnp.where(t != 0, row, 0.0)
    rows_ref[...] = row.reshape(rows_ref.shape)


def _extract_call(am, atten, text3):
    B, L, _ = atten.shape
    grid_spec = pltpu.PrefetchScalarGridSpec(
        num_scalar_prefetch=1,
        grid=(B,),
        in_specs=[
            pl.BlockSpec((1, 8, L), lambda b, am_s: (b, am_s[b] // 8, 0)),
            pl.BlockSpec((1, 1, L), lambda b, am_s: (b, 0, 0)),
        ],
        out_specs=pl.BlockSpec((1, 1, L), lambda b, am_s: (b, 0, 0)),
    )
    return pl.pallas_call(
        _extract_kernel,
        grid_spec=grid_spec,
        out_shape=jax.ShapeDtypeStruct((B, 1, L), jnp.float32),
    )(am, atten, text3)


def _topk_kernel(rows_ref, idx_ref, *, k, k_pad):
    v = rows_ref[...]
    B, L = v.shape
    ix = jax.lax.broadcasted_iota(jnp.int32, (B, L), 1)
    ci = jax.lax.broadcasted_iota(jnp.int32, (1, L), 1)

    kk = 2
    while kk <= L:
        j = kk // 2
        while j >= 1:
            bit = (ci & j) != 0
            desc = (ci & kk) == 0
            if kk == L:
                desc = jnp.full_like(bit, True)
            pv = jnp.where(bit, jnp.roll(v, j, axis=1), jnp.roll(v, -j, axis=1))
            px = jnp.where(bit, jnp.roll(ix, j, axis=1), jnp.roll(ix, -j, axis=1))
            lo_v = jnp.where(bit, pv, v)
            hi_v = jnp.where(bit, v, pv)
            lo_i = jnp.where(bit, px, ix)
            hi_i = jnp.where(bit, ix, px)
            good = (lo_v > hi_v) | ((lo_v == hi_v) & (lo_i < hi_i))
            swap = jnp.logical_xor(good, desc)
            v = jnp.where(swap, pv, v)
            ix = jnp.where(swap, px, ix)
            j //= 2
        kk *= 2

    out = ix[:, :k_pad]
    cpad = jax.lax.broadcasted_iota(jnp.int32, (1, k_pad), 1)
    idx_ref[...] = jnp.where(cpad < k, out, -1)


def _topk_call(rows, k, k_pad):
    B, L = rows.shape
    return pl.pallas_call(
        functools.partial(_topk_kernel, k=k, k_pad=k_pad),
        out_shape=jax.ShapeDtypeStruct((B, k_pad), jnp.int32),
    )(rows)


def _gather_stats_kernel(feat_ref, tk_ref, xn_ref, g_ref, s_ref,
                         g_acc, s_acc):
    b = pl.program_id(0)
    nb = pl.num_programs(0)
    f = feat_ref[0]
    tk = tk_ref[0]
    L, D = f.shape
    kp = tk.shape[-1]
    cc = jax.lax.broadcasted_iota(jnp.int32, (L, kp), 0)
    pt = (cc == tk).astype(jnp.float32)
    f_hi = f.astype(jnp.bfloat16).astype(jnp.float32)
    f_lo = f - f_hi
    x = (jax.lax.dot_general(pt, f_hi, (((0,), (0,)), ((), ())),
                             preferred_element_type=jnp.float32)
         + jax.lax.dot_general(pt, f_lo, (((0,), (0,)), ((), ())),
                               preferred_element_type=jnp.float32))
    norm = jnp.sqrt(jnp.sum(x * x, axis=1, keepdims=True)) + 1e-8
    xn = x / norm
    xn_ref[0] = xn.astype(jnp.bfloat16)

    g_step = jax.lax.dot_general(xn, xn, (((0,), (0,)), ((), ())),
                                 preferred_element_type=jnp.float32)
    s_step = jnp.sum(xn, axis=0, keepdims=True)

    @pl.when(b == 0)
    def _():
        g_acc[...] = jnp.zeros_like(g_acc)
        s_acc[...] = jnp.zeros_like(s_acc)

    g_acc[...] += g_step
    s_acc[...] += s_step

    @pl.when(b == nb - 1)
    def _():
        g_ref[...] = g_acc[...]
        s_ref[...] = s_acc[...]


def _gather_stats_call(features, tk3, k_pad):
    B, L, D = features.shape
    grid = (B,)
    return pl.pallas_call(
        _gather_stats_kernel,
        grid=grid,
        in_specs=[
            pl.BlockSpec((1, L, D), lambda b: (b, 0, 0)),
            pl.BlockSpec((1, 1, k_pad), lambda b: (b, 0, 0)),
        ],
        out_specs=[
            pl.BlockSpec((1, k_pad, D), lambda b: (b, 0, 0)),
            pl.BlockSpec((D, D), lambda b: (0, 0)),
            pl.BlockSpec((1, D), lambda b: (0, 0)),
        ],
        out_shape=[
            jax.ShapeDtypeStruct((B, k_pad, D), jnp.bfloat16),
            jax.ShapeDtypeStruct((D, D), jnp.float32),
            jax.ShapeDtypeStruct((1, D), jnp.float32),
        ],
        scratch_shapes=[
            pltpu.VMEM((D, D), jnp.float32),
            pltpu.VMEM((1, D), jnp.float32),
        ],
    )(features, tk3)


def _final_kernel(xn_ref, g_ref, s_ref, lw_ref, lb_ref, w1_ref, b1_ref,
                  gam_ref, bet_ref, w2_ref, b2_ref, out_ref,
                  scale_scr, shift_scr, *, n_rows):
    b = pl.program_id(0)

    k = out_ref.shape[1]

    @pl.when(b == 0)
    def _():
        w1 = w1_ref[...].astype(jnp.float32)
        s = s_ref[...] / n_rows
        g = g_ref[...] / n_rows
        mean_wx = jax.lax.dot_general(s, w1, (((1,), (1,)), ((), ())),
                                      preferred_element_type=jnp.float32)
        m = jax.lax.dot_general(w1, g, (((1,), (0,)), ((), ())),
                                preferred_element_type=jnp.float32)
        ones = jnp.ones((1, m.shape[1]), jnp.float32)
        e = jax.lax.dot_general(ones, m * w1, (((1,), (1,)), ((), ())),
                                preferred_element_type=jnp.float32)
        var = e - mean_wx * mean_wx
        mean_h = mean_wx + b1_ref[...]
        scale = gam_ref[...] * jax.lax.rsqrt(var + 1e-5)
        scale_scr[...] = scale
        shift_scr[...] = bet_ref[...] - mean_h * scale

    x = xn_ref[0]
    cap = jax.lax.dot_general(x, lw_ref[...], (((1,), (1,)), ((), ())),
                              preferred_element_type=jnp.float32) + lb_ref[...]
    h = jax.lax.dot_general(x, w1_ref[...], (((1,), (1,)), ((), ())),
                            preferred_element_type=jnp.float32) + b1_ref[...]
    h = h * scale_scr[...] + shift_scr[...]
    h = jnp.maximum(h, 0.0).astype(jnp.bfloat16)
    out = jax.lax.dot_general(h, w2_ref[...], (((1,), (1,)), ((), ())),
                              preferred_element_type=jnp.float32) + b2_ref[...]
    out_ref[0] = (out + cap)[:k]


def _final_call(xn, g, s, lw, lb, w1, b1, gam, bet, w2, b2, n_rows, k):
    B, k_pad, D = xn.shape
    DE = lw.shape[0]
    H = w1.shape[0]
    grid = (B,)
    return pl.pallas_call(
        functools.partial(_final_kernel, n_rows=float(n_rows)),
        grid=grid,
        in_specs=[
            pl.BlockSpec((1, k_pad, D), lambda b: (b, 0, 0)),
            pl.BlockSpec((D, D), lambda b: (0, 0)),
            pl.BlockSpec((1, D), lambda b: (0, 0)),
            pl.BlockSpec((DE, D), lambda b: (0, 0)),
            pl.BlockSpec((1, DE), lambda b: (0, 0)),
            pl.BlockSpec((H, D), lambda b: (0, 0)),
            pl.BlockSpec((1, H), lambda b: (0, 0)),
            pl.BlockSpec((1, H), lambda b: (0, 0)),
            pl.BlockSpec((1, H), lambda b: (0, 0)),
            pl.BlockSpec((DE, H), lambda b: (0, 0)),
            pl.BlockSpec((1, DE), lambda b: (0, 0)),
        ],
        out_specs=pl.BlockSpec((1, k, DE), lambda b: (b, 0, 0)),
        out_shape=jax.ShapeDtypeStruct((B, k, DE), jnp.float32),
        scratch_shapes=[
            pltpu.VMEM((1, H), jnp.float32),
            pltpu.VMEM((1, H), jnp.float32),
        ],
    )(xn, g, s, lw, lb, w1, b1, gam, bet, w2, b2)


def kernel(features, text, atten, linear_W, linear_b, mlp_W1, mlp_b1,
           bn_gamma, bn_beta, mlp_W2, mlp_b2):
    B, L, D = features.shape
    k = int((atten.shape[1] - 2) * _RATIO)
    k_pad = ((k + 7) // 8) * 8

    am = _argmax_call(text).reshape(B)
    rows = _extract_call(am, atten, text.reshape(B, 1, L)).reshape(B, L)
    tk = _topk_call(rows, k, k_pad)
    tk3 = tk.reshape(B, 1, k_pad)
    xn, g, s = _gather_stats_call(features, tk3, k_pad)
    bf = jnp.bfloat16
    out = _final_call(xn, g, s, linear_W.astype(bf), linear_b.reshape(1, -1),
                      mlp_W1.astype(bf), mlp_b1.reshape(1, -1),
                      bn_gamma.reshape(1, -1), bn_beta.reshape(1, -1),
                      mlp_W2.astype(bf), mlp_b2.reshape(1, -1),
                      n_rows=B * k, k=k)
    return out


# speedup vs baseline: 1.4955x; 1.1976x over previous
import functools
import jax
import jax.numpy as jnp
from jax.experimental import pallas as pl
from jax.experimental.pallas import tpu as pltpu

_RATIO = 0.3


def _argmax_kernel(text_ref, am_ref):
    t = text_ref[...]
    b, l = t.shape
    m = jnp.max(t, axis=1, keepdims=True)
    iota = jax.lax.broadcasted_iota(jnp.int32, (b, l), 1)
    am_ref[...] = jnp.min(jnp.where(t == m, iota, l), axis=1, keepdims=True)


def _argmax_call(text):
    B, L = text.shape
    return pl.pallas_call(
        _argmax_kernel,
        out_shape=jax.ShapeDtypeStruct((B, 1), jnp.int32),
    )(text)


def _extract_kernel(am_ref, atten_ref, text_ref, rows_ref):
    b = pl.program_id(0)
    a = am_ref[b]
    row = atten_ref[0, pl.ds(a % 8, 1), :]
    t = text_ref[...].reshape(1, text_ref.shape[-1])
    iota = jax.lax.broadcasted_iota(jnp.int32, row.shape, 1)
    row = jnp.where(iota == a, -1.0, row)
    row = jnp.where(t != 0, row, 0.0)
    rows_ref[...] = row.reshape(rows_ref.shape)


def _extract_call(am, atten, text3):
    B, L, _ = atten.shape
    grid_spec = pltpu.PrefetchScalarGridSpec(
        num_scalar_prefetch=1,
        grid=(B,),
        in_specs=[
            pl.BlockSpec((1, 8, L), lambda b, am_s: (b, am_s[b] // 8, 0)),
            pl.BlockSpec((1, 1, L), lambda b, am_s: (b, 0, 0)),
        ],
        out_specs=pl.BlockSpec((1, 1, L), lambda b, am_s: (b, 0, 0)),
    )
    return pl.pallas_call(
        _extract_kernel,
        grid_spec=grid_spec,
        out_shape=jax.ShapeDtypeStruct((B, 1, L), jnp.float32),
    )(am, atten, text3)


def _topk_kernel(rows_ref, idx_ref, *, k, k_pad):
    v = rows_ref[...]
    B, L = v.shape
    ix = jax.lax.broadcasted_iota(jnp.int32, (B, L), 1)
    ci = jax.lax.broadcasted_iota(jnp.int32, (1, L), 1)

    kk = 2
    while kk <= L:
        j = kk // 2
        while j >= 1:
            bit = (ci & j) != 0
            desc = (ci & kk) == 0
            if kk == L:
                desc = jnp.full_like(bit, True)
            pv = jnp.where(bit, jnp.roll(v, j, axis=1), jnp.roll(v, -j, axis=1))
            px = jnp.where(bit, jnp.roll(ix, j, axis=1), jnp.roll(ix, -j, axis=1))
            lo_v = jnp.where(bit, pv, v)
            hi_v = jnp.where(bit, v, pv)
            lo_i = jnp.where(bit, px, ix)
            hi_i = jnp.where(bit, ix, px)
            good = (lo_v > hi_v) | ((lo_v == hi_v) & (lo_i < hi_i))
            swap = jnp.logical_xor(good, desc)
            v = jnp.where(swap, pv, v)
            ix = jnp.where(swap, px, ix)
            j //= 2
        kk *= 2

    out = ix[:, :k_pad]
    cpad = jax.lax.broadcasted_iota(jnp.int32, (1, k_pad), 1)
    idx_ref[...] = jnp.where(cpad < k, out, -1)


def _topk_call(rows, k, k_pad):
    B, L = rows.shape
    return pl.pallas_call(
        functools.partial(_topk_kernel, k=k, k_pad=k_pad),
        out_shape=jax.ShapeDtypeStruct((B, k_pad), jnp.int32),
    )(rows)


def _gather_stats_kernel(feat_ref, tk_ref, xn_ref, g_ref, s_ref,
                         g_acc, s_acc):
    b = pl.program_id(0)
    nb = pl.num_programs(0)
    f = feat_ref[0]
    tk = tk_ref[0]
    L, D = f.shape
    kp = tk.shape[-1]
    cc = jax.lax.broadcasted_iota(jnp.int32, (L, kp), 0)
    pt = (cc == tk).astype(jnp.float32)
    f_hi = f.astype(jnp.bfloat16).astype(jnp.float32)
    f_lo = f - f_hi
    x = (jax.lax.dot_general(pt, f_hi, (((0,), (0,)), ((), ())),
                             preferred_element_type=jnp.float32)
         + jax.lax.dot_general(pt, f_lo, (((0,), (0,)), ((), ())),
                               preferred_element_type=jnp.float32))
    norm = jnp.sqrt(jnp.sum(x * x, axis=1, keepdims=True)) + 1e-8
    xn = x / norm
    xn_ref[0] = xn.astype(jnp.bfloat16)

    g_step = jax.lax.dot_general(xn, xn, (((0,), (0,)), ((), ())),
                                 preferred_element_type=jnp.float32)
    s_step = jnp.sum(xn, axis=0, keepdims=True)

    @pl.when(b == 0)
    def _():
        g_acc[...] = jnp.zeros_like(g_acc)
        s_acc[...] = jnp.zeros_like(s_acc)

    g_acc[...] += g_step
    s_acc[...] += s_step

    @pl.when(b == nb - 1)
    def _():
        g_ref[...] = g_acc[...]
        s_ref[...] = s_acc[...]


def _gather_stats_call(features, tk3, k_pad):
    B, L, D = features.shape
    grid = (B,)
    return pl.pallas_call(
        _gather_stats_kernel,
        grid=grid,
        in_specs=[
            pl.BlockSpec((1, L, D), lambda b: (b, 0, 0)),
            pl.BlockSpec((1, 1, k_pad), lambda b: (b, 0, 0)),
        ],
        out_specs=[
            pl.BlockSpec((1, k_pad, D), lambda b: (b, 0, 0)),
            pl.BlockSpec((D, D), lambda b: (0, 0)),
            pl.BlockSpec((1, D), lambda b: (0, 0)),
        ],
        out_shape=[
            jax.ShapeDtypeStruct((B, k_pad, D), jnp.bfloat16),
            jax.ShapeDtypeStruct((D, D), jnp.float32),
            jax.ShapeDtypeStruct((1, D), jnp.float32),
        ],
        scratch_shapes=[
            pltpu.VMEM((D, D), jnp.float32),
            pltpu.VMEM((1, D), jnp.float32),
        ],
    )(features, tk3)


def _final_kernel(xn_ref, g_ref, s_ref, lw_ref, lb_ref, w1_ref, b1_ref,
                  gam_ref, bet_ref, w2_ref, b2_ref, out_ref,
                  scale_scr, shift_scr, *, n_rows):
    b = pl.program_id(0)

    k = out_ref.shape[0]

    @pl.when(b == 0)
    def _():
        w1 = w1_ref[...].astype(jnp.float32)
        s = s_ref[...] / n_rows
        g = g_ref[...] / n_rows
        mean_wx = jax.lax.dot_general(s, w1, (((1,), (1,)), ((), ())),
                                      preferred_element_type=jnp.float32)
        m = jax.lax.dot_general(w1, g, (((1,), (0,)), ((), ())),
                                preferred_element_type=jnp.float32)
        ones = jnp.ones((1, m.shape[1]), jnp.float32)
        e = jax.lax.dot_general(ones, m * w1, (((1,), (1,)), ((), ())),
                                preferred_element_type=jnp.float32)
        var = e - mean_wx * mean_wx
        mean_h = mean_wx + b1_ref[...]
        scale = gam_ref[...] * jax.lax.rsqrt(var + 1e-5)
        scale_scr[...] = scale
        shift_scr[...] = bet_ref[...] - mean_h * scale

    for bb in range(xn_ref.shape[0]):
        x = xn_ref[bb]
        cap = jax.lax.dot_general(x, lw_ref[...], (((1,), (1,)), ((), ())),
                                  preferred_element_type=jnp.float32) + lb_ref[...]
        h = jax.lax.dot_general(x, w1_ref[...], (((1,), (1,)), ((), ())),
                                preferred_element_type=jnp.float32) + b1_ref[...]
        h = h * scale_scr[...] + shift_scr[...]
        h = jnp.maximum(h, 0.0).astype(jnp.bfloat16)
        out = jax.lax.dot_general(h, w2_ref[...], (((1,), (1,)), ((), ())),
                                  preferred_element_type=jnp.float32) + b2_ref[...]
        out_ref[:, bb, :] = (out + cap)[:k]


def _final_call(xn, g, s, lw, lb, w1, b1, gam, bet, w2, b2, n_rows, k):
    B, k_pad, D = xn.shape
    DE = lw.shape[0]
    H = w1.shape[0]
    bblk = 8
    grid = (B // bblk,)
    return pl.pallas_call(
        functools.partial(_final_kernel, n_rows=float(n_rows)),
        grid=grid,
        in_specs=[
            pl.BlockSpec((bblk, k_pad, D), lambda b: (b, 0, 0)),
            pl.BlockSpec((D, D), lambda b: (0, 0)),
            pl.BlockSpec((1, D), lambda b: (0, 0)),
            pl.BlockSpec((DE, D), lambda b: (0, 0)),
            pl.BlockSpec((1, DE), lambda b: (0, 0)),
            pl.BlockSpec((H, D), lambda b: (0, 0)),
            pl.BlockSpec((1, H), lambda b: (0, 0)),
            pl.BlockSpec((1, H), lambda b: (0, 0)),
            pl.BlockSpec((1, H), lambda b: (0, 0)),
            pl.BlockSpec((DE, H), lambda b: (0, 0)),
            pl.BlockSpec((1, DE), lambda b: (0, 0)),
        ],
        out_specs=pl.BlockSpec((k, bblk, DE), lambda b: (0, b, 0)),
        out_shape=jax.ShapeDtypeStruct((k, B, DE), jnp.float32),
        scratch_shapes=[
            pltpu.VMEM((1, H), jnp.float32),
            pltpu.VMEM((1, H), jnp.float32),
        ],
    )(xn, g, s, lw, lb, w1, b1, gam, bet, w2, b2)


def kernel(features, text, atten, linear_W, linear_b, mlp_W1, mlp_b1,
           bn_gamma, bn_beta, mlp_W2, mlp_b2):
    B, L, D = features.shape
    k = int((atten.shape[1] - 2) * _RATIO)
    k_pad = ((k + 7) // 8) * 8

    am = _argmax_call(text).reshape(B)
    rows = _extract_call(am, atten, text.reshape(B, 1, L)).reshape(B, L)
    tk = _topk_call(rows, k, k_pad)
    tk3 = tk.reshape(B, 1, k_pad)
    xn, g, s = _gather_stats_call(features, tk3, k_pad)
    bf = jnp.bfloat16
    out = _final_call(xn, g, s, linear_W.astype(bf), linear_b.reshape(1, -1),
                      mlp_W1.astype(bf), mlp_b1.reshape(1, -1),
                      bn_gamma.reshape(1, -1), bn_beta.reshape(1, -1),
                      mlp_W2.astype(bf), mlp_b2.reshape(1, -1),
                      n_rows=B * k, k=k)
    return jnp.transpose(out, (1, 0, 2))


# speedup vs baseline: 2.0744x; 1.3870x over previous
import functools
import jax
import jax.numpy as jnp
from jax.experimental import pallas as pl
from jax.experimental.pallas import tpu as pltpu

_RATIO = 0.3


def _argmax_kernel(text_ref, am_ref):
    t = text_ref[...]
    b, l = t.shape
    m = jnp.max(t, axis=1, keepdims=True)
    iota = jax.lax.broadcasted_iota(jnp.int32, (b, l), 1)
    am_ref[...] = jnp.min(jnp.where(t == m, iota, l), axis=1, keepdims=True)


def _argmax_call(text):
    B, L = text.shape
    return pl.pallas_call(
        _argmax_kernel,
        out_shape=jax.ShapeDtypeStruct((B, 1), jnp.int32),
    )(text)


_EXBLK = 8


def _extract_kernel(am_ref, *refs):
    atten_refs = refs[:_EXBLK]
    text_ref, rows_ref = refs[_EXBLK], refs[_EXBLK + 1]
    g = pl.program_id(0)
    for j in range(_EXBLK):
        a = am_ref[g * _EXBLK + j]
        row = atten_refs[j][0, pl.ds(a % 8, 1), :]
        t = text_ref[j]
        iota = jax.lax.broadcasted_iota(jnp.int32, row.shape, 1)
        row = jnp.where(iota == a, -1.0, row)
        row = jnp.where(t != 0, row, 0.0)
        rows_ref[j] = row


def _extract_call(am, atten, text3):
    B, L, _ = atten.shape

    def mk_spec(j):
        return pl.BlockSpec(
            (1, 8, L),
            lambda b, am_s, j=j: (b * _EXBLK + j, am_s[b * _EXBLK + j] // 8, 0))

    grid_spec = pltpu.PrefetchScalarGridSpec(
        num_scalar_prefetch=1,
        grid=(B // _EXBLK,),
        in_specs=[mk_spec(j) for j in range(_EXBLK)] + [
            pl.BlockSpec((_EXBLK, 1, L), lambda b, am_s: (b, 0, 0)),
        ],
        out_specs=pl.BlockSpec((_EXBLK, 1, L), lambda b, am_s: (b, 0, 0)),
    )
    return pl.pallas_call(
        _extract_kernel,
        grid_spec=grid_spec,
        out_shape=jax.ShapeDtypeStruct((B, 1, L), jnp.float32),
    )(am, *([atten] * _EXBLK), text3)


def _topk_kernel(rows_ref, idx_ref, *, k, k_pad):
    v = rows_ref[...]
    B, L = v.shape
    ix = jax.lax.broadcasted_iota(jnp.int32, (B, L), 1)
    ci = jax.lax.broadcasted_iota(jnp.int32, (1, L), 1)

    kk = 2
    while kk <= L:
        j = kk // 2
        while j >= 1:
            bit = (ci & j) != 0
            desc = (ci & kk) == 0
            if kk == L:
                desc = jnp.full_like(bit, True)
            pv = jnp.where(bit, jnp.roll(v, j, axis=1), jnp.roll(v, -j, axis=1))
            px = jnp.where(bit, jnp.roll(ix, j, axis=1), jnp.roll(ix, -j, axis=1))
            lo_v = jnp.where(bit, pv, v)
            hi_v = jnp.where(bit, v, pv)
            lo_i = jnp.where(bit, px, ix)
            hi_i = jnp.where(bit, ix, px)
            good = (lo_v > hi_v) | ((lo_v == hi_v) & (lo_i < hi_i))
            swap = jnp.logical_xor(good, desc)
            v = jnp.where(swap, pv, v)
            ix = jnp.where(swap, px, ix)
            j //= 2
        kk *= 2

    out = ix[:, :k_pad]
    cpad = jax.lax.broadcasted_iota(jnp.int32, (1, k_pad), 1)
    idx_ref[...] = jnp.where(cpad < k, out, -1)


def _topk_call(rows, k, k_pad):
    B, L = rows.shape
    return pl.pallas_call(
        functools.partial(_topk_kernel, k=k, k_pad=k_pad),
        out_shape=jax.ShapeDtypeStruct((B, k_pad), jnp.int32),
    )(rows)


def _gather_stats_kernel(feat_ref, tk_ref, xn_ref, g_ref, s_ref,
                         g_acc, s_acc):
    b = pl.program_id(0)
    nb = pl.num_programs(0)
    L, D = feat_ref.shape[1], feat_ref.shape[2]
    kp = tk_ref.shape[-1]

    g_step = None
    s_step = None
    for bb in range(feat_ref.shape[0]):
        f = feat_ref[bb]
        tk = tk_ref[bb]
        cc = jax.lax.broadcasted_iota(jnp.int32, (L, kp), 0)
        pt = (cc == tk).astype(jnp.float32)
        f_hi = f.astype(jnp.bfloat16).astype(jnp.float32)
        f_lo = f - f_hi
        x = (jax.lax.dot_general(pt, f_hi, (((0,), (0,)), ((), ())),
                                 preferred_element_type=jnp.float32)
             + jax.lax.dot_general(pt, f_lo, (((0,), (0,)), ((), ())),
                                   preferred_element_type=jnp.float32))
        norm = jnp.sqrt(jnp.sum(x * x, axis=1, keepdims=True)) + 1e-8
        xn = x / norm
        xn_ref[bb] = xn.astype(jnp.bfloat16)
        g_bb = jax.lax.dot_general(xn, xn, (((0,), (0,)), ((), ())),
                                   preferred_element_type=jnp.float32)
        s_bb = jnp.sum(xn, axis=0, keepdims=True)
        g_step = g_bb if g_step is None else g_step + g_bb
        s_step = s_bb if s_step is None else s_step + s_bb

    @pl.when(b == 0)
    def _():
        g_acc[...] = jnp.zeros_like(g_acc)
        s_acc[...] = jnp.zeros_like(s_acc)

    g_acc[...] += g_step
    s_acc[...] += s_step

    @pl.when(b == nb - 1)
    def _():
        g_ref[...] = g_acc[...]
        s_ref[...] = s_acc[...]


def _gather_stats_call(features, tk3, k_pad):
    B, L, D = features.shape
    bblk = 4
    grid = (B // bblk,)
    return pl.pallas_call(
        _gather_stats_kernel,
        grid=grid,
        in_specs=[
            pl.BlockSpec((bblk, L, D), lambda b: (b, 0, 0)),
            pl.BlockSpec((bblk, 1, k_pad), lambda b: (b, 0, 0)),
        ],
        out_specs=[
            pl.BlockSpec((bblk, k_pad, D), lambda b: (b, 0, 0)),
            pl.BlockSpec((D, D), lambda b: (0, 0)),
            pl.BlockSpec((1, D), lambda b: (0, 0)),
        ],
        out_shape=[
            jax.ShapeDtypeStruct((B, k_pad, D), jnp.bfloat16),
            jax.ShapeDtypeStruct((D, D), jnp.float32),
            jax.ShapeDtypeStruct((1, D), jnp.float32),
        ],
        scratch_shapes=[
            pltpu.VMEM((D, D), jnp.float32),
            pltpu.VMEM((1, D), jnp.float32),
        ],
    )(features, tk3)


def _final_kernel(xn_ref, g_ref, s_ref, lw_ref, lb_ref, w1_ref, b1_ref,
                  gam_ref, bet_ref, w2_ref, b2_ref, out_ref,
                  scale_scr, shift_scr, *, n_rows):
    b = pl.program_id(0)

    k = out_ref.shape[0]

    @pl.when(b == 0)
    def _():
        w1 = w1_ref[...].astype(jnp.float32)
        s = s_ref[...] / n_rows
        g = g_ref[...] / n_rows
        mean_wx = jax.lax.dot_general(s, w1, (((1,), (1,)), ((), ())),
                                      preferred_element_type=jnp.float32)
        m = jax.lax.dot_general(w1, g, (((1,), (0,)), ((), ())),
                                preferred_element_type=jnp.float32)
        ones = jnp.ones((1, m.shape[1]), jnp.float32)
        e = jax.lax.dot_general(ones, m * w1, (((1,), (1,)), ((), ())),
                                preferred_element_type=jnp.float32)
        var = e - mean_wx * mean_wx
        mean_h = mean_wx + b1_ref[...]
        scale = gam_ref[...] * jax.lax.rsqrt(var + 1e-5)
        scale_scr[...] = scale
        shift_scr[...] = bet_ref[...] - mean_h * scale

    for bb in range(xn_ref.shape[0]):
        x = xn_ref[bb]
        cap = jax.lax.dot_general(x, lw_ref[...], (((1,), (1,)), ((), ())),
                                  preferred_element_type=jnp.float32) + lb_ref[...]
        h = jax.lax.dot_general(x, w1_ref[...], (((1,), (1,)), ((), ())),
                                preferred_element_type=jnp.float32) + b1_ref[...]
        h = h * scale_scr[...] + shift_scr[...]
        h = jnp.maximum(h, 0.0).astype(jnp.bfloat16)
        out = jax.lax.dot_general(h, w2_ref[...], (((1,), (1,)), ((), ())),
                                  preferred_element_type=jnp.float32) + b2_ref[...]
        out_ref[:, bb, :] = (out + cap)[:k]


def _final_call(xn, g, s, lw, lb, w1, b1, gam, bet, w2, b2, n_rows, k):
    B, k_pad, D = xn.shape
    DE = lw.shape[0]
    H = w1.shape[0]
    bblk = 8
    grid = (B // bblk,)
    return pl.pallas_call(
        functools.partial(_final_kernel, n_rows=float(n_rows)),
        grid=grid,
        in_specs=[
            pl.BlockSpec((bblk, k_pad, D), lambda b: (b, 0, 0)),
            pl.BlockSpec((D, D), lambda b: (0, 0)),
            pl.BlockSpec((1, D), lambda b: (0, 0)),
            pl.BlockSpec((DE, D), lambda b: (0, 0)),
            pl.BlockSpec((1, DE), lambda b: (0, 0)),
            pl.BlockSpec((H, D), lambda b: (0, 0)),
            pl.BlockSpec((1, H), lambda b: (0, 0)),
            pl.BlockSpec((1, H), lambda b: (0, 0)),
            pl.BlockSpec((1, H), lambda b: (0, 0)),
            pl.BlockSpec((DE, H), lambda b: (0, 0)),
            pl.BlockSpec((1, DE), lambda b: (0, 0)),
        ],
        out_specs=pl.BlockSpec((k, bblk, DE), lambda b: (0, b, 0)),
        out_shape=jax.ShapeDtypeStruct((k, B, DE), jnp.float32),
        scratch_shapes=[
            pltpu.VMEM((1, H), jnp.float32),
            pltpu.VMEM((1, H), jnp.float32),
        ],
    )(xn, g, s, lw, lb, w1, b1, gam, bet, w2, b2)


def kernel(features, text, atten, linear_W, linear_b, mlp_W1, mlp_b1,
           bn_gamma, bn_beta, mlp_W2, mlp_b2):
    B, L, D = features.shape
    k = int((atten.shape[1] - 2) * _RATIO)
    k_pad = ((k + 7) // 8) * 8

    am = _argmax_call(text).reshape(B)
    rows = _extract_call(am, atten, text.reshape(B, 1, L)).reshape(B, L)
    tk = _topk_call(rows, k, k_pad)
    tk3 = tk.reshape(B, 1, k_pad)
    xn, g, s = _gather_stats_call(features, tk3, k_pad)
    bf = jnp.bfloat16
    out = _final_call(xn, g, s, linear_W.astype(bf), linear_b.reshape(1, -1),
                      mlp_W1.astype(bf), mlp_b1.reshape(1, -1),
                      bn_gamma.reshape(1, -1), bn_beta.reshape(1, -1),
                      mlp_W2.astype(bf), mlp_b2.reshape(1, -1),
                      n_rows=B * k, k=k)
    return jnp.transpose(out, (1, 0, 2))


# speedup vs baseline: 2.0823x; 1.0038x over previous
import functools
import jax
import jax.numpy as jnp
from jax.experimental import pallas as pl
from jax.experimental.pallas import tpu as pltpu

_RATIO = 0.3


def _argmax_kernel(text_ref, am_ref):
    t = text_ref[...]
    b, l = t.shape
    m = jnp.max(t, axis=1, keepdims=True)
    iota = jax.lax.broadcasted_iota(jnp.int32, (b, l), 1)
    am_ref[...] = jnp.min(jnp.where(t == m, iota, l), axis=1, keepdims=True)


def _argmax_call(text):
    B, L = text.shape
    return pl.pallas_call(
        _argmax_kernel,
        out_shape=jax.ShapeDtypeStruct((B, 1), jnp.int32),
    )(text)


_EXBLK = 8


def _extract_kernel(am_ref, *refs):
    atten_refs = refs[:_EXBLK]
    text_ref, rows_ref = refs[_EXBLK], refs[_EXBLK + 1]
    g = pl.program_id(0)
    for j in range(_EXBLK):
        a = am_ref[g * _EXBLK + j]
        row = atten_refs[j][0, pl.ds(a % 8, 1), :]
        t = text_ref[j]
        iota = jax.lax.broadcasted_iota(jnp.int32, row.shape, 1)
        row = jnp.where(iota == a, -1.0, row)
        row = jnp.where(t != 0, row, 0.0)
        rows_ref[j] = row


def _extract_call(am, atten, text3):
    B, L, _ = atten.shape

    def mk_spec(j):
        return pl.BlockSpec(
            (1, 8, L),
            lambda b, am_s, j=j: (b * _EXBLK + j, am_s[b * _EXBLK + j] // 8, 0))

    grid_spec = pltpu.PrefetchScalarGridSpec(
        num_scalar_prefetch=1,
        grid=(B // _EXBLK,),
        in_specs=[mk_spec(j) for j in range(_EXBLK)] + [
            pl.BlockSpec((_EXBLK, 1, L), lambda b, am_s: (b, 0, 0)),
        ],
        out_specs=pl.BlockSpec((_EXBLK, 1, L), lambda b, am_s: (b, 0, 0)),
    )
    return pl.pallas_call(
        _extract_kernel,
        grid_spec=grid_spec,
        out_shape=jax.ShapeDtypeStruct((B, 1, L), jnp.float32),
    )(am, *([atten] * _EXBLK), text3)


def _topk_kernel(rows_ref, idx_ref, *, k, k_pad):
    v = rows_ref[...]
    B, L = v.shape
    ix = jax.lax.broadcasted_iota(jnp.int32, (B, L), 1)
    ci = jax.lax.broadcasted_iota(jnp.int32, (1, L), 1)

    kk = 2
    while kk <= L:
        j = kk // 2
        while j >= 1:
            bit = (ci & j) != 0
            desc = (ci & kk) == 0
            if kk == L:
                desc = jnp.full_like(bit, True)
            pv = jnp.where(bit, jnp.roll(v, j, axis=1), jnp.roll(v, -j, axis=1))
            px = jnp.where(bit, jnp.roll(ix, j, axis=1), jnp.roll(ix, -j, axis=1))
            lo_v = jnp.where(bit, pv, v)
            hi_v = jnp.where(bit, v, pv)
            lo_i = jnp.where(bit, px, ix)
            hi_i = jnp.where(bit, ix, px)
            good = (lo_v > hi_v) | ((lo_v == hi_v) & (lo_i < hi_i))
            swap = jnp.logical_xor(good, desc)
            v = jnp.where(swap, pv, v)
            ix = jnp.where(swap, px, ix)
            j //= 2
        kk *= 2

    out = ix[:, :k_pad]
    cpad = jax.lax.broadcasted_iota(jnp.int32, (1, k_pad), 1)
    idx_ref[...] = jnp.where(cpad < k, out, -1)


def _topk_call(rows, k, k_pad):
    B, L = rows.shape
    return pl.pallas_call(
        functools.partial(_topk_kernel, k=k, k_pad=k_pad),
        out_shape=jax.ShapeDtypeStruct((B, k_pad), jnp.int32),
    )(rows)


def _gather_stats_kernel(feat_ref, tk_ref, xn_ref, g_ref, s_ref,
                         g_acc, s_acc):
    b = pl.program_id(0)
    nb = pl.num_programs(0)
    L, D = feat_ref.shape[1], feat_ref.shape[2]
    kp = tk_ref.shape[-1]

    g_step = None
    s_step = None
    for bb in range(feat_ref.shape[0]):
        f = feat_ref[bb]
        tk = tk_ref[bb]
        cc = jax.lax.broadcasted_iota(jnp.int32, (L, kp), 0)
        pt = (cc == tk).astype(jnp.float32)
        f_hi = f.astype(jnp.bfloat16).astype(jnp.float32)
        f_lo = f - f_hi
        x = (jax.lax.dot_general(pt, f_hi, (((0,), (0,)), ((), ())),
                                 preferred_element_type=jnp.float32)
             + jax.lax.dot_general(pt, f_lo, (((0,), (0,)), ((), ())),
                                   preferred_element_type=jnp.float32))
        norm = jnp.sqrt(jnp.sum(x * x, axis=1, keepdims=True)) + 1e-8
        xn = x / norm
        xn_ref[bb] = xn.astype(jnp.bfloat16)
        g_bb = jax.lax.dot_general(xn, xn, (((0,), (0,)), ((), ())),
                                   preferred_element_type=jnp.float32)
        s_bb = jnp.sum(xn, axis=0, keepdims=True)
        g_step = g_bb if g_step is None else g_step + g_bb
        s_step = s_bb if s_step is None else s_step + s_bb

    @pl.when(b == 0)
    def _():
        g_acc[...] = jnp.zeros_like(g_acc)
        s_acc[...] = jnp.zeros_like(s_acc)

    g_acc[...] += g_step
    s_acc[...] += s_step

    @pl.when(b == nb - 1)
    def _():
        g_ref[...] = g_acc[...]
        s_ref[...] = s_acc[...]


def _gather_stats_call(features, tk3, k_pad):
    B, L, D = features.shape
    bblk = 8
    grid = (B // bblk,)
    return pl.pallas_call(
        _gather_stats_kernel,
        grid=grid,
        in_specs=[
            pl.BlockSpec((bblk, L, D), lambda b: (b, 0, 0)),
            pl.BlockSpec((bblk, 1, k_pad), lambda b: (b, 0, 0)),
        ],
        out_specs=[
            pl.BlockSpec((bblk, k_pad, D), lambda b: (b, 0, 0)),
            pl.BlockSpec((D, D), lambda b: (0, 0)),
            pl.BlockSpec((1, D), lambda b: (0, 0)),
        ],
        out_shape=[
            jax.ShapeDtypeStruct((B, k_pad, D), jnp.bfloat16),
            jax.ShapeDtypeStruct((D, D), jnp.float32),
            jax.ShapeDtypeStruct((1, D), jnp.float32),
        ],
        scratch_shapes=[
            pltpu.VMEM((D, D), jnp.float32),
            pltpu.VMEM((1, D), jnp.float32),
        ],
    )(features, tk3)


def _final_kernel(xn_ref, g_ref, s_ref, lw_ref, lb_ref, w1_ref, b1_ref,
                  gam_ref, bet_ref, w2_ref, b2_ref, out_ref,
                  scale_scr, shift_scr, *, n_rows):
    b = pl.program_id(0)

    k = out_ref.shape[0]

    @pl.when(b == 0)
    def _():
        w1 = w1_ref[...].astype(jnp.float32)
        s = s_ref[...] / n_rows
        g = g_ref[...] / n_rows
        mean_wx = jax.lax.dot_general(s, w1, (((1,), (1,)), ((), ())),
                                      preferred_element_type=jnp.float32)
        m = jax.lax.dot_general(w1, g, (((1,), (0,)), ((), ())),
                                preferred_element_type=jnp.float32)
        ones = jnp.ones((1, m.shape[1]), jnp.float32)
        e = jax.lax.dot_general(ones, m * w1, (((1,), (1,)), ((), ())),
                                preferred_element_type=jnp.float32)
        var = e - mean_wx * mean_wx
        mean_h = mean_wx + b1_ref[...]
        scale = gam_ref[...] * jax.lax.rsqrt(var + 1e-5)
        scale_scr[...] = scale
        shift_scr[...] = bet_ref[...] - mean_h * scale

    for bb in range(xn_ref.shape[0]):
        x = xn_ref[bb]
        cap = jax.lax.dot_general(x, lw_ref[...], (((1,), (1,)), ((), ())),
                                  preferred_element_type=jnp.float32) + lb_ref[...]
        h = jax.lax.dot_general(x, w1_ref[...], (((1,), (1,)), ((), ())),
                                preferred_element_type=jnp.float32) + b1_ref[...]
        h = h * scale_scr[...] + shift_scr[...]
        h = jnp.maximum(h, 0.0).astype(jnp.bfloat16)
        out = jax.lax.dot_general(h, w2_ref[...], (((1,), (1,)), ((), ())),
                                  preferred_element_type=jnp.float32) + b2_ref[...]
        out_ref[:, bb, :] = (out + cap)[:k]


def _final_call(xn, g, s, lw, lb, w1, b1, gam, bet, w2, b2, n_rows, k):
    B, k_pad, D = xn.shape
    DE = lw.shape[0]
    H = w1.shape[0]
    bblk = 8
    grid = (B // bblk,)
    return pl.pallas_call(
        functools.partial(_final_kernel, n_rows=float(n_rows)),
        grid=grid,
        in_specs=[
            pl.BlockSpec((bblk, k_pad, D), lambda b: (b, 0, 0)),
            pl.BlockSpec((D, D), lambda b: (0, 0)),
            pl.BlockSpec((1, D), lambda b: (0, 0)),
            pl.BlockSpec((DE, D), lambda b: (0, 0)),
            pl.BlockSpec((1, DE), lambda b: (0, 0)),
            pl.BlockSpec((H, D), lambda b: (0, 0)),
            pl.BlockSpec((1, H), lambda b: (0, 0)),
            pl.BlockSpec((1, H), lambda b: (0, 0)),
            pl.BlockSpec((1, H), lambda b: (0, 0)),
            pl.BlockSpec((DE, H), lambda b: (0, 0)),
            pl.BlockSpec((1, DE), lambda b: (0, 0)),
        ],
        out_specs=pl.BlockSpec((k, bblk, DE), lambda b: (0, b, 0)),
        out_shape=jax.ShapeDtypeStruct((k, B, DE), jnp.float32),
        scratch_shapes=[
            pltpu.VMEM((1, H), jnp.float32),
            pltpu.VMEM((1, H), jnp.float32),
        ],
    )(xn, g, s, lw, lb, w1, b1, gam, bet, w2, b2)


def kernel(features, text, atten, linear_W, linear_b, mlp_W1, mlp_b1,
           bn_gamma, bn_beta, mlp_W2, mlp_b2):
    B, L, D = features.shape
    k = int((atten.shape[1] - 2) * _RATIO)
    k_pad = ((k + 7) // 8) * 8

    am = _argmax_call(text).reshape(B)
    rows = _extract_call(am, atten, text.reshape(B, 1, L)).reshape(B, L)
    tk = _topk_call(rows, k, k_pad)
    tk3 = tk.reshape(B, 1, k_pad)
    xn, g, s = _gather_stats_call(features, tk3, k_pad)
    bf = jnp.bfloat16
    out = _final_call(xn, g, s, linear_W.astype(bf), linear_b.reshape(1, -1),
                      mlp_W1.astype(bf), mlp_b1.reshape(1, -1),
                      bn_gamma.reshape(1, -1), bn_beta.reshape(1, -1),
                      mlp_W2.astype(bf), mlp_b2.reshape(1, -1),
                      n_rows=B * k, k=k)
    return jnp.transpose(out, (1, 0, 2))
